# baseline (speedup 1.0000x reference)
import functools

import jax
import jax.numpy as jnp
from jax import lax
from jax.experimental import pallas as pl
from jax.experimental.pallas import tpu as pltpu
from jax.experimental.pallas import tpu_sc as plsc

L = 16
NC = 2
NS = 16
NW = NC * NS
BATCH = 16384
BPW = BATCH // NW
D = 64
NU = 100000
NI = 1000000
PU = (NU + 127) // 128
PI = (NI + 127) // 128
PPT_U = (PU + NW - 1) // NW
PPT_I = (PI + NW - 1) // NW
NGRP_U = (PPT_U + 7) // 8
NGRP_I = (PPT_I + 7) // 8
NCHUNK = BATCH // L
NBUF = 4
STAGE_SLOTS = 128
DRAIN_AT = 96

_IOTA = lambda: lax.iota(jnp.int32, L)


def _dynsel(vec, i):
    return jnp.sum(jnp.where(_IOTA() == i, vec, 0))


def _side(idx_hbm, tab_hbm, out_hbm, n_panels, ppt, ngrp,
          wid, idx_v, hits_v, bufs, stage_v, sems, semo):
    p0 = wid * ppt
    myppt = jnp.minimum(ppt, n_panels - p0)
    iota = _IOTA()

    pltpu.sync_copy(idx_hbm, idx_v)

    def scan_body(k4, cur):
        for j in range(4):
            k = k4 * 4 + j
            v = idx_v[pl.ds(k * L, L)]
            pan = v >> 7
            m = (pan >= p0) & (pan < p0 + myppt)
            p_l = pan - p0
            rl = v & 127
            b = iota + k * L
            val = (p_l << 21) | (rl << 14) | b
            plsc.store_compressed(hits_v.at[pl.ds(cur, L)], val, mask=m)
            cur = cur + plsc.all_reduce_population_count(m)[0]
        return cur

    nh = lax.fori_loop(0, NCHUNK // 4, scan_body, jnp.int32(0))
    nch = (nh + L - 1) // L

    bounds_lo = jnp.zeros((L,), jnp.int32)
    bounds_hi = jnp.zeros((L,), jnp.int32)
    cur = jnp.int32(0)
    for g in range(ngrp):
        def dist_body(k4, c, g=g):
            for j in range(4):
                k = k4 * 4 + j
                hv = hits_v[pl.ds(k * L, L)]
                lidx = iota + k * L
                m = (lidx < nh) & (((hv >> 24) & 31) == g)
                plsc.store_compressed(idx_v.at[pl.ds(c, L)], hv, mask=m)
                c = c + plsc.all_reduce_population_count(m)[0]
            return c

        cur = lax.fori_loop(0, (nch + 3) // 4, dist_body, cur)
        if g < L:
            bounds_lo = jnp.where(iota == g, cur, bounds_lo)
        else:
            bounds_hi = jnp.where(iota == (g - L), cur, bounds_hi)

    nf = (myppt + 1) // 2

    def unit_q(t):
        return jnp.minimum(p0 + 2 * t, n_panels - 2)

    def start_unit(t, buf, sem):
        pltpu.async_copy(tab_hbm.at[:, pl.ds(unit_q(t) * 128, 256)], buf, sem)

    def wait_unit(buf, sem):
        pltpu.make_async_copy(tab_hbm.at[:, pl.ds(0, 256)], buf, sem).wait()

    def drain_rows(n):
        def one(_, c):
            pltpu.make_async_copy(out_hbm.at[pl.ds(0, D)],
                                  stage_v.at[pl.ds(0, D)], semo).wait()
            return c
        lax.fori_loop(0, n, one, jnp.int32(0))

    def process_unit(t, buf, w):
        q = unit_q(t)
        g = (2 * t) >> 3
        start = _dynsel(bounds_lo, g - 1) + _dynsel(bounds_hi, g - 1 - L)
        end = _dynsel(bounds_lo, g) + _dynsel(bounds_hi, g - L)
        c_lo = start >> 4
        c_hi = (end + L - 1) >> 4

        def chunk_body(k, w):
            hv = idx_v[pl.ds(k * L, L)]
            lidx = iota + k * L
            pa = (hv >> 21) + p0
            m = (lidx >= start) & (lidx < end) & ((pa == q) | (pa == q + 1))
            col_v = ((hv >> 14) & 127) + (pa - q) * 128
            mi = jnp.where(m, 1, 0)
            exc = plsc.cumsum(mi) - mi
            for lk in range(L):
                m_k = mi[lk]

                @pl.when(m_k > 0)
                def _(lk=lk):
                    val = hv[lk]
                    b = val & 0x3FFF
                    col = col_v[lk]
                    wk = w + exc[lk]

                    def feat(qq, c):
                        fcol = plsc.load_gather(
                            buf,
                            [iota + qq * L, jnp.full((L,), col, jnp.int32)])
                        stage_v[pl.ds(wk * D + qq * L, L)] = fcol
                        return c
                    lax.fori_loop(0, D // L, feat, jnp.int32(0))
                    pltpu.async_copy(
                        stage_v.at[pl.ds(wk * D, D)],
                        out_hbm.at[pl.ds(b * D, D)], semo)

            w = w + plsc.all_reduce_population_count(m)[0]

            @pl.when(w >= DRAIN_AT)
            def _():
                drain_rows(w)

            return jnp.where(w >= DRAIN_AT, 0, w)

        return lax.fori_loop(c_lo, c_hi, chunk_body, w)

    for u in range(NBUF - 1):
        @pl.when(u < nf)
        def _(u=u):
            start_unit(jnp.int32(u), bufs[u], sems[u])

    def ring_body(t4, w):
        for u in range(NBUF):
            t = t4 * NBUF + u

            def do(w, u=u, t=t):
                @pl.when(t + NBUF - 1 < nf)
                def _():
                    start_unit(t + NBUF - 1, bufs[(u + NBUF - 1) % NBUF],
                               sems[(u + NBUF - 1) % NBUF])
                wait_unit(bufs[u], sems[u])
                return process_unit(t, bufs[u], w)

            w = lax.cond(t < nf, do, lambda w: w, w)
        return w

    w = lax.fori_loop(0, (nf + NBUF - 1) // NBUF, ring_body, jnp.int32(0))
    drain_rows(w)


def _phase1_body(uidx_hbm, iidx_hbm, uft_hbm, ift_hbm, ug_hbm, ig_hbm,
                 idx_v, hits_v, buf0, buf1, buf2, buf3, stage_v,
                 sem0, sem1, sem2, sem3, semo):
    wid = lax.axis_index("s") * NC + lax.axis_index("c")
    bufs = (buf0, buf1, buf2, buf3)
    sems = (sem0, sem1, sem2, sem3)
    _side(uidx_hbm, uft_hbm, ug_hbm, PU, PPT_U, NGRP_U, wid,
          idx_v, hits_v, bufs, stage_v, sems, semo)
    _side(iidx_hbm, ift_hbm, ig_hbm, PI, PPT_I, NGRP_I, wid,
          idx_v, hits_v, bufs, stage_v, sems, semo)


def _phase2_body(ug_hbm, ig_hbm, out_hbm, ug_v, ig_v, out_v, sem):
    wid = lax.axis_index("s") * NC + lax.axis_index("c")
    base = wid * BPW
    pltpu.async_copy(ug_hbm.at[pl.ds(base, BPW), :], ug_v, sem).wait()
    pltpu.async_copy(ig_hbm.at[pl.ds(base, BPW), :], ig_v, sem).wait()
    lane = _IOTA()

    def chunk(g, carry):
        rows = lane + g * L
        accs = [jnp.zeros((L,), jnp.float32) for _ in range(4)]
        for c in range(D):
            col = jnp.full((L,), c, jnp.int32)
            u = plsc.load_gather(ug_v, [rows, col])
            v = plsc.load_gather(ig_v, [rows, col])
            accs[c % 4] = accs[c % 4] + u * v
        out_v[pl.ds(g * L, L)] = (accs[0] + accs[1]) + (accs[2] + accs[3])
        return carry

    lax.fori_loop(0, BPW // L, chunk, 0)
    pltpu.sync_copy(out_v, out_hbm.at[pl.ds(base, BPW)])


def kernel(user_idx, item_idx, user_factors, item_factors):
    user_idx = user_idx.astype(jnp.int32)
    item_idx = item_idx.astype(jnp.int32)
    mesh = plsc.VectorSubcoreMesh(core_axis_name="c", subcore_axis_name="s")

    phase1 = functools.partial(
        pl.kernel,
        out_type=(jax.ShapeDtypeStruct((BATCH * D,), jnp.float32),
                  jax.ShapeDtypeStruct((BATCH * D,), jnp.float32)),
        mesh=mesh,
        compiler_params=pltpu.CompilerParams(
            needs_layout_passes=False, use_tc_tiling_on_sc=True),
        scratch_types=[
            pltpu.VMEM((BATCH,), jnp.int32),
            pltpu.VMEM((BATCH + 64,), jnp.int32),
            pltpu.VMEM((D, 256), jnp.float32),
            pltpu.VMEM((D, 256), jnp.float32),
            pltpu.VMEM((D, 256), jnp.float32),
            pltpu.VMEM((D, 256), jnp.float32),
            pltpu.VMEM((STAGE_SLOTS * D,), jnp.float32),
            pltpu.SemaphoreType.DMA,
            pltpu.SemaphoreType.DMA,
            pltpu.SemaphoreType.DMA,
            pltpu.SemaphoreType.DMA,
            pltpu.SemaphoreType.DMA,
        ],
    )(_phase1_body)

    ug, ig = phase1(user_idx, item_idx, user_factors.T, item_factors.T)

    phase2 = functools.partial(
        pl.kernel,
        out_type=jax.ShapeDtypeStruct((BATCH,), jnp.float32),
        mesh=mesh,
        compiler_params=pltpu.CompilerParams(
            needs_layout_passes=False, use_tc_tiling_on_sc=False),
        scratch_types=[
            pltpu.VMEM((BPW, D), jnp.float32),
            pltpu.VMEM((BPW, D), jnp.float32),
            pltpu.VMEM((BPW,), jnp.float32),
            pltpu.SemaphoreType.DMA,
        ],
    )(_phase2_body)

    return phase2(ug.reshape(BATCH, D), ig.reshape(BATCH, D))


# speedup vs baseline: 1.2385x; 1.2385x over previous
import functools

import jax
import jax.numpy as jnp
from jax import lax
from jax.experimental import pallas as pl
from jax.experimental.pallas import tpu as pltpu
from jax.experimental.pallas import tpu_sc as plsc

L = 16
NC = 2
NS = 16
NW = NC * NS
BATCH = 16384
BPW = BATCH // NW
D = 64
NU = 100000
NI = 1000000
PU = (NU + 127) // 128
PI = (NI + 127) // 128
PPT_U = (PU + NW - 1) // NW
PPT_I = (PI + NW - 1) // NW
NGRP_U = (PPT_U + 7) // 8
NGRP_I = (PPT_I + 7) // 8
NCHUNK = BATCH // L
NBUF = 2
STAGE_SLOTS = 128
DRAIN_AT = 96

_IOTA = lambda: lax.iota(jnp.int32, L)


def _dynsel(vec, i):
    return jnp.sum(jnp.where(_IOTA() == i, vec, 0))


def _side(idx_hbm, tab_hbm, out_hbm, n_panels, ppt, ngrp,
          wid, idx_v, hits_v, bufs, stage_v, sems, semo):
    p0 = wid * ppt
    myppt = jnp.minimum(ppt, n_panels - p0)
    iota = _IOTA()

    pltpu.sync_copy(idx_hbm, idx_v)

    def scan_body(k4, cur):
        for j in range(4):
            k = k4 * 4 + j
            v = idx_v[pl.ds(k * L, L)]
            pan = v >> 7
            m = (pan >= p0) & (pan < p0 + myppt)
            p_l = pan - p0
            rl = v & 127
            b = iota + k * L
            val = (p_l << 21) | (rl << 14) | b
            plsc.store_compressed(hits_v.at[pl.ds(cur, L)], val, mask=m)
            cur = cur + plsc.all_reduce_population_count(m)[0]
        return cur

    nh = lax.fori_loop(0, NCHUNK // 4, scan_body, jnp.int32(0))
    nch = (nh + L - 1) // L

    bounds_lo = jnp.zeros((L,), jnp.int32)
    bounds_hi = jnp.zeros((L,), jnp.int32)
    cur = jnp.int32(0)
    for g in range(ngrp):
        def dist_body(k4, c, g=g):
            for j in range(4):
                k = k4 * 4 + j
                hv = hits_v[pl.ds(k * L, L)]
                lidx = iota + k * L
                m = (lidx < nh) & (((hv >> 24) & 31) == g)
                plsc.store_compressed(idx_v.at[pl.ds(c, L)], hv, mask=m)
                c = c + plsc.all_reduce_population_count(m)[0]
            return c

        cur = lax.fori_loop(0, (nch + 3) // 4, dist_body, cur)
        if g < L:
            bounds_lo = jnp.where(iota == g, cur, bounds_lo)
        else:
            bounds_hi = jnp.where(iota == (g - L), cur, bounds_hi)

    nf = (myppt + 3) // 4

    def unit_q(t):
        return jnp.minimum(p0 + 4 * t, n_panels - 4)

    def start_unit(t, buf, sem):
        q = unit_q(t)
        for c1 in range(8):
            pltpu.async_copy(
                tab_hbm.at[pl.ds(c1 * 8, 8), pl.ds(q * 128, 512)],
                buf.at[pl.ds(c1 * 8, 8), :], sem)

    def wait_unit(buf, sem):
        pltpu.make_async_copy(tab_hbm.at[:, pl.ds(0, 512)], buf, sem).wait()

    def drain_rows(n):
        def one(_, c):
            pltpu.make_async_copy(out_hbm.at[pl.ds(0, D)],
                                  stage_v.at[pl.ds(0, D)], semo).wait()
            return c
        lax.fori_loop(0, n, one, jnp.int32(0))

    def process_unit(t, buf, w):
        q = unit_q(t)
        g = (4 * t) >> 3
        start = _dynsel(bounds_lo, g - 1) + _dynsel(bounds_hi, g - 1 - L)
        end = _dynsel(bounds_lo, g) + _dynsel(bounds_hi, g - L)
        c_lo = start >> 4
        c_hi = (end + L - 1) >> 4

        def chunk_body(k, w):
            hv = idx_v[pl.ds(k * L, L)]
            lidx = iota + k * L
            pa = (hv >> 21) + p0
            m = (lidx >= start) & (lidx < end) & (pa >= q) & (pa < q + 4)
            col_v = ((hv >> 14) & 127) + (pa - q) * 128
            mi = jnp.where(m, 1, 0)
            exc = plsc.cumsum(mi) - mi
            for lk in range(L):
                m_k = mi[lk]

                @pl.when(m_k > 0)
                def _(lk=lk):
                    val = hv[lk]
                    b = val & 0x3FFF
                    col = col_v[lk]
                    wk = w + exc[lk]

                    def feat(qq, c):
                        fcol = plsc.load_gather(
                            buf,
                            [iota + qq * L, jnp.full((L,), col, jnp.int32)])
                        stage_v[pl.ds(wk * D + qq * L, L)] = fcol
                        return c
                    lax.fori_loop(0, D // L, feat, jnp.int32(0))
                    pltpu.async_copy(
                        stage_v.at[pl.ds(wk * D, D)],
                        out_hbm.at[pl.ds(b * D, D)], semo)

            w = w + plsc.all_reduce_population_count(m)[0]

            @pl.when(w >= DRAIN_AT)
            def _():
                drain_rows(w)

            return jnp.where(w >= DRAIN_AT, 0, w)

        return lax.fori_loop(c_lo, c_hi, chunk_body, w)

    for u in range(NBUF - 1):
        @pl.when(u < nf)
        def _(u=u):
            start_unit(jnp.int32(u), bufs[u], sems[u])

    def ring_body(t4, w):
        for u in range(NBUF):
            t = t4 * NBUF + u

            def do(w, u=u, t=t):
                @pl.when(t + NBUF - 1 < nf)
                def _():
                    start_unit(t + NBUF - 1, bufs[(u + NBUF - 1) % NBUF],
                               sems[(u + NBUF - 1) % NBUF])
                wait_unit(bufs[u], sems[u])
                return process_unit(t, bufs[u], w)

            w = lax.cond(t < nf, do, lambda w: w, w)
        return w

    w = lax.fori_loop(0, (nf + NBUF - 1) // NBUF, ring_body, jnp.int32(0))
    drain_rows(w)


def _phase1_body(uidx_hbm, iidx_hbm, uft_hbm, ift_hbm, ug_hbm, ig_hbm,
                 idx_v, hits_v, buf0, buf1, stage_v,
                 sem0, sem1, semo):
    wid = lax.axis_index("s") * NC + lax.axis_index("c")
    bufs = (buf0, buf1)
    sems = (sem0, sem1)
    _side(uidx_hbm, uft_hbm, ug_hbm, PU, PPT_U, NGRP_U, wid,
          idx_v, hits_v, bufs, stage_v, sems, semo)
    _side(iidx_hbm, ift_hbm, ig_hbm, PI, PPT_I, NGRP_I, wid,
          idx_v, hits_v, bufs, stage_v, sems, semo)


def _phase2_body(ug_hbm, ig_hbm, out_hbm, ug_v, ig_v, out_v, sem):
    wid = lax.axis_index("s") * NC + lax.axis_index("c")
    base = wid * BPW
    pltpu.async_copy(ug_hbm.at[pl.ds(base, BPW), :], ug_v, sem).wait()
    pltpu.async_copy(ig_hbm.at[pl.ds(base, BPW), :], ig_v, sem).wait()
    lane = _IOTA()

    def chunk(g, carry):
        rows = lane + g * L
        accs = [jnp.zeros((L,), jnp.float32) for _ in range(4)]
        for c in range(D):
            col = jnp.full((L,), c, jnp.int32)
            u = plsc.load_gather(ug_v, [rows, col])
            v = plsc.load_gather(ig_v, [rows, col])
            accs[c % 4] = accs[c % 4] + u * v
        out_v[pl.ds(g * L, L)] = (accs[0] + accs[1]) + (accs[2] + accs[3])
        return carry

    lax.fori_loop(0, BPW // L, chunk, 0)
    pltpu.sync_copy(out_v, out_hbm.at[pl.ds(base, BPW)])


def kernel(user_idx, item_idx, user_factors, item_factors):
    user_idx = user_idx.astype(jnp.int32)
    item_idx = item_idx.astype(jnp.int32)
    mesh = plsc.VectorSubcoreMesh(core_axis_name="c", subcore_axis_name="s")

    phase1 = functools.partial(
        pl.kernel,
        out_type=(jax.ShapeDtypeStruct((BATCH * D,), jnp.float32),
                  jax.ShapeDtypeStruct((BATCH * D,), jnp.float32)),
        mesh=mesh,
        compiler_params=pltpu.CompilerParams(
            needs_layout_passes=False, use_tc_tiling_on_sc=True),
        scratch_types=[
            pltpu.VMEM((BATCH,), jnp.int32),
            pltpu.VMEM((BATCH + 64,), jnp.int32),
            pltpu.VMEM((D, 512), jnp.float32),
            pltpu.VMEM((D, 512), jnp.float32),
            pltpu.VMEM((STAGE_SLOTS * D,), jnp.float32),
            pltpu.SemaphoreType.DMA,
            pltpu.SemaphoreType.DMA,
            pltpu.SemaphoreType.DMA,
        ],
    )(_phase1_body)

    ug, ig = phase1(user_idx, item_idx, user_factors.T, item_factors.T)

    phase2 = functools.partial(
        pl.kernel,
        out_type=jax.ShapeDtypeStruct((BATCH,), jnp.float32),
        mesh=mesh,
        compiler_params=pltpu.CompilerParams(
            needs_layout_passes=False, use_tc_tiling_on_sc=False),
        scratch_types=[
            pltpu.VMEM((BPW, D), jnp.float32),
            pltpu.VMEM((BPW, D), jnp.float32),
            pltpu.VMEM((BPW,), jnp.float32),
            pltpu.SemaphoreType.DMA,
        ],
    )(_phase2_body)

    return phase2(ug.reshape(BATCH, D), ig.reshape(BATCH, D))
